# baseline (speedup 1.0000x reference)
import functools

import jax
import jax.numpy as jnp
from jax.experimental import pallas as pl
from jax.experimental.pallas import tpu as pltpu
from jax.experimental.pallas import tpu_sc as plsc

_CAPACITY = 0.5


def _score_kernel(x_ref, w_ref, b_ref, o_ref):
    s = jax.lax.dot_general(
        w_ref[...], x_ref[...],
        dimension_numbers=(((1,), (1,)), ((), ())),
        preferred_element_type=jnp.float32,
    )
    o_ref[...] = (s + b_ref[0, 0])[None]


def _sc_select(scores, k):
    B_, T_ = scores.shape
    nch = T_ // 16
    mesh = plsc.VectorSubcoreMesh(core_axis_name="c", subcore_axis_name="s")

    @functools.partial(
        pl.kernel,
        out_type=(jax.ShapeDtypeStruct((B_, T_), jnp.int32),
                  jax.ShapeDtypeStruct((B_, T_), jnp.float32)),
        mesh=mesh,
        scratch_types=[pltpu.VMEM((T_,), jnp.float32),
                       pltpu.VMEM((T_,), jnp.int32),
                       pltpu.VMEM((T_,), jnp.int32),
                       pltpu.VMEM((T_,), jnp.float32)],
    )
    def body(s_hbm, mask_hbm, w_hbm, row_v, key_v, m_v, w_v):
        c = jax.lax.axis_index("c")
        si = jax.lax.axis_index("s")
        wid = si * 2 + c

        @pl.when(wid < B_)
        def _():
            pltpu.sync_copy(s_hbm.at[wid], row_v)

            def lane_sum(acc):
                tot = acc[0]
                for j in range(1, 16):
                    tot = tot + acc[j]
                return tot

            def enc(i, carry):
                v = row_v[pl.ds(i * 16, 16)]
                bb = jax.lax.bitcast_convert_type(v, jnp.int32)
                mag = bb & jnp.int32(0x7FFFFFFF)
                key_v[pl.ds(i * 16, 16)] = jnp.where(
                    bb >= 0, bb, jnp.int32(-1) - mag)
                w_v[pl.ds(i * 16, 16)] = 1.0 / (1.0 + jnp.exp(-v))
                return carry
            jax.lax.fori_loop(0, nch, enc, 0)

            def count_ge(cand):
                cv = jnp.full((16,), cand, jnp.int32)

                def chunk(i, acc):
                    kk = key_v[pl.ds(i * 16, 16)]
                    return acc + jnp.where(kk >= cv, jnp.int32(1), jnp.int32(0))
                acc = jax.lax.fori_loop(0, nch, chunk,
                                        jnp.zeros((16,), jnp.int32))
                return lane_sum(acc)

            thr0 = jnp.where(count_ge(jnp.int32(0)) >= k,
                             jnp.int32(0), jnp.int32(-(2 ** 31)))

            def vbit(i, thr):
                cand = thr + (jnp.int32(1) << (jnp.int32(30) - i))
                return jnp.where(count_ge(cand) >= k, cand, thr)
            thr = jax.lax.fori_loop(0, 31, vbit, thr0)

            cnt_gt = count_ge(thr + 1)
            need = k - cnt_gt

            def count_eq_below(m):
                tv = jnp.full((16,), thr, jnp.int32)
                mv = jnp.full((16,), m, jnp.int32)
                base = jax.lax.iota(jnp.int32, 16)

                def chunk(i, acc):
                    kk = key_v[pl.ds(i * 16, 16)]
                    p = base + i * 16
                    hit = (kk == tv) & (p < mv)
                    return acc + jnp.where(hit, jnp.int32(1), jnp.int32(0))
                acc = jax.lax.fori_loop(0, nch, chunk,
                                        jnp.zeros((16,), jnp.int32))
                return lane_sum(acc)

            cnt_eq = count_eq_below(jnp.int32(T_))

            def tie_search():
                def step(_, lh):
                    lo, hi = lh
                    mid = (lo + hi) // 2
                    ge = count_eq_below(mid) >= need
                    return (jnp.where(ge, lo, mid), jnp.where(ge, mid, hi))
                _, hi = jax.lax.fori_loop(
                    0, max(1, T_.bit_length()), step,
                    (jnp.int32(0), jnp.int32(T_)))
                return hi

            bigm = jax.lax.cond(need == cnt_eq,
                                lambda: jnp.int32(T_), tie_search)

            def emit(i, carry):
                kk = key_v[pl.ds(i * 16, 16)]
                tv = jnp.full((16,), thr, jnp.int32)
                mv = jnp.full((16,), bigm, jnp.int32)
                p = jax.lax.iota(jnp.int32, 16) + i * 16
                m = (kk > tv) | ((kk == tv) & (p < mv))
                m_v[pl.ds(i * 16, 16)] = jnp.where(m, jnp.int32(1), jnp.int32(0))
                return carry
            jax.lax.fori_loop(0, nch, emit, 0)

            pltpu.sync_copy(m_v, mask_hbm.at[wid])
            pltpu.sync_copy(w_v, w_hbm.at[wid])

    return body(scores)


def kernel(x, W, b):
    B, T, D = x.shape
    k = max(1, int(T * _CAPACITY))

    Tt = 1024
    rows = (B * T) // Tt
    xr = x.reshape(B * T, D)
    b2 = b.reshape(1, 1)

    scores = pl.pallas_call(
        _score_kernel,
        grid=(rows,),
        in_specs=[
            pl.BlockSpec((Tt, D), lambda i: (i, 0)),
            pl.BlockSpec((1, D), lambda i: (0, 0)),
            pl.BlockSpec((1, 1), lambda i: (0, 0)),
        ],
        out_specs=pl.BlockSpec((1, 1, Tt), lambda i: (i, 0, 0)),
        out_shape=jax.ShapeDtypeStruct((rows, 1, Tt), jnp.float32),
    )(xr, W, b2).reshape(B, T)

    mask_i, weights = _sc_select(scores, k)
    return (mask_i.astype(jnp.bool_), weights)


# speedup vs baseline: 1.4882x; 1.4882x over previous
import functools

import jax
import jax.numpy as jnp
from jax.experimental import pallas as pl
from jax.experimental.pallas import tpu as pltpu

_CAPACITY = 0.5


def _select(s, k):
    Bn, S, L = s.shape
    weights = jax.nn.sigmoid(s)

    b32 = jax.lax.bitcast_convert_type(s, jnp.int32)
    mag = b32 & jnp.int32(0x7FFFFFFF)
    keys = jnp.where(b32 >= 0, b32, jnp.int32(-1) - mag)

    def count_ge(c):
        return jnp.sum((keys >= c).astype(jnp.int32), axis=(1, 2),
                       keepdims=True)

    thr = jnp.where(count_ge(jnp.int32(0)) >= k,
                    jnp.int32(0), jnp.int32(-(2**31)))
    for hi_bit in range(30, 0, -2):
        q = jnp.int32(1 << (hi_bit - 1))
        d1 = (count_ge(thr + q) >= k).astype(jnp.int32)
        d2 = (count_ge(thr + 2 * q) >= k).astype(jnp.int32)
        d3 = (count_ge(thr + 3 * q) >= k).astype(jnp.int32)
        thr = thr + q * (d1 + d2 + d3)
    thr = jnp.where(count_ge(thr + 1) >= k, thr + 1, thr)

    gt = keys > thr
    eq = keys == thr
    cnt_gt = jnp.sum(gt.astype(jnp.int32), axis=(1, 2), keepdims=True)
    cnt_eq = jnp.sum(eq.astype(jnp.int32), axis=(1, 2), keepdims=True)
    need = k - cnt_gt

    pos = (jax.lax.broadcasted_iota(jnp.int32, (Bn, S, L), 1) * L
           + jax.lax.broadcasted_iota(jnp.int32, (Bn, S, L), 2))
    T = S * L

    def tie_search():
        lo = jnp.zeros((Bn, 1, 1), jnp.int32)
        hi = jnp.full((Bn, 1, 1), T, jnp.int32)
        for _ in range((T.bit_length() + 1) // 2 + 1):
            w = hi - lo
            m1, m2, m3 = lo + w // 4, lo + w // 2, lo + (3 * w) // 4
            c1 = jnp.sum((eq & (pos < m1)).astype(jnp.int32), axis=(1, 2),
                         keepdims=True) >= need
            c2 = jnp.sum((eq & (pos < m2)).astype(jnp.int32), axis=(1, 2),
                         keepdims=True) >= need
            c3 = jnp.sum((eq & (pos < m3)).astype(jnp.int32), axis=(1, 2),
                         keepdims=True) >= need
            hi = jnp.where(c1, m1, jnp.where(c2, m2, jnp.where(c3, m3, hi)))
            lo = jnp.where(~c3, m3, jnp.where(~c2, m2, jnp.where(~c1, m1, lo)))
        return hi

    no_ties = jnp.all(need == cnt_eq)
    hi = jax.lax.cond(no_ties,
                      lambda: jnp.full((Bn, 1, 1), T, jnp.int32),
                      tie_search)
    return gt | (eq & (pos < hi)), weights


_NS = 2
_TT = 256


def _fused_kernel(*refs, k, nsteps, sub, ns):
    x_refs = refs[:ns]
    w_ref, b_ref, mask_ref, wout_ref, sc_ref = refs[ns:]
    i = pl.program_id(0)
    for j in range(ns):
        s = jax.lax.dot_general(
            w_ref[...], x_refs[j][...],
            dimension_numbers=(((1,), (1,)), ((), ())),
            preferred_element_type=jnp.float32,
        ) + b_ref[0, 0]
        a = i * ns + j
        sc_ref[a // sub, a % sub, :] = s[0]

    @pl.when(i == nsteps - 1)
    def _():
        mask, weights = _select(sc_ref[...], k)
        mask_ref[...] = mask
        wout_ref[...] = weights


def kernel(x, W, b):
    B, T, D = x.shape
    k = max(1, int(T * _CAPACITY))

    Tt, ns = _TT, _NS
    nsteps = (B * T) // (Tt * ns)
    sub = T // Tt
    xr = x.reshape(B * T, D)
    b2 = b.reshape(1, 1)

    def mk_spec(j):
        return pl.BlockSpec((Tt, D), lambda i: (i * ns + j, 0))

    mask3, w3 = pl.pallas_call(
        functools.partial(_fused_kernel, k=k, nsteps=nsteps, sub=sub, ns=ns),
        grid=(nsteps,),
        in_specs=[mk_spec(j) for j in range(ns)] + [
            pl.BlockSpec((1, D), lambda i: (0, 0)),
            pl.BlockSpec((1, 1), lambda i: (0, 0)),
        ],
        out_specs=(
            pl.BlockSpec((B, sub, Tt), lambda i: (0, 0, 0)),
            pl.BlockSpec((B, sub, Tt), lambda i: (0, 0, 0)),
        ),
        out_shape=(
            jax.ShapeDtypeStruct((B, sub, Tt), jnp.bool_),
            jax.ShapeDtypeStruct((B, sub, Tt), jnp.float32),
        ),
        scratch_shapes=[pltpu.VMEM((B, sub, Tt), jnp.float32)],
    )(*([xr] * ns), W, b2)
    return (mask3.reshape(B, T), w3.reshape(B, T))


# speedup vs baseline: 1.4895x; 1.0009x over previous
import functools

import jax
import jax.numpy as jnp
from jax.experimental import pallas as pl
from jax.experimental.pallas import tpu as pltpu

_CAPACITY = 0.5


def _select(s, k):
    Bn, S, L = s.shape
    weights = jax.nn.sigmoid(s)

    b32 = jax.lax.bitcast_convert_type(s, jnp.int32)
    mag = b32 & jnp.int32(0x7FFFFFFF)
    keys = jnp.where(b32 >= 0, b32, jnp.int32(-1) - mag)

    def count_ge(c):
        return jnp.sum((keys >= c).astype(jnp.int32), axis=(1, 2),
                       keepdims=True)

    thr = jnp.where(count_ge(jnp.int32(0)) >= k,
                    jnp.int32(0), jnp.int32(-(2**31)))
    for hi_bit in range(30, 0, -3):
        q = jnp.int32(1 << (hi_bit - 2))
        step = jnp.zeros_like(thr)
        for m in range(1, 8):
            step = step + (count_ge(thr + m * q) >= k).astype(jnp.int32)
        thr = thr + q * step
    thr = jnp.where(count_ge(thr + 1) >= k, thr + 1, thr)

    gt = keys > thr
    eq = keys == thr
    cnt_gt = jnp.sum(gt.astype(jnp.int32), axis=(1, 2), keepdims=True)
    cnt_eq = jnp.sum(eq.astype(jnp.int32), axis=(1, 2), keepdims=True)
    need = k - cnt_gt

    pos = (jax.lax.broadcasted_iota(jnp.int32, (Bn, S, L), 1) * L
           + jax.lax.broadcasted_iota(jnp.int32, (Bn, S, L), 2))
    T = S * L

    def tie_search():
        lo = jnp.zeros((Bn, 1, 1), jnp.int32)
        hi = jnp.full((Bn, 1, 1), T, jnp.int32)
        for _ in range((T.bit_length() + 1) // 2 + 1):
            w = hi - lo
            m1, m2, m3 = lo + w // 4, lo + w // 2, lo + (3 * w) // 4
            c1 = jnp.sum((eq & (pos < m1)).astype(jnp.int32), axis=(1, 2),
                         keepdims=True) >= need
            c2 = jnp.sum((eq & (pos < m2)).astype(jnp.int32), axis=(1, 2),
                         keepdims=True) >= need
            c3 = jnp.sum((eq & (pos < m3)).astype(jnp.int32), axis=(1, 2),
                         keepdims=True) >= need
            hi = jnp.where(c1, m1, jnp.where(c2, m2, jnp.where(c3, m3, hi)))
            lo = jnp.where(~c3, m3, jnp.where(~c2, m2, jnp.where(~c1, m1, lo)))
        return hi

    no_ties = jnp.all(need == cnt_eq)
    hi = jax.lax.cond(no_ties,
                      lambda: jnp.full((Bn, 1, 1), T, jnp.int32),
                      tie_search)
    return gt | (eq & (pos < hi)), weights


_NS = 2
_TT = 512


def _fused_kernel(*refs, k, nsteps, sub, ns):
    x_refs = refs[:ns]
    w_ref, b_ref, mask_ref, wout_ref, sc_ref = refs[ns:]
    i = pl.program_id(0)
    for j in range(ns):
        s = jax.lax.dot_general(
            w_ref[...], x_refs[j][...],
            dimension_numbers=(((1,), (1,)), ((), ())),
            preferred_element_type=jnp.float32,
        ) + b_ref[0, 0]
        a = i * ns + j
        sc_ref[a // sub, a % sub, :] = s[0]

    @pl.when(i == nsteps - 1)
    def _():
        mask, weights = _select(sc_ref[...], k)
        mask_ref[...] = mask
        wout_ref[...] = weights


def kernel(x, W, b):
    B, T, D = x.shape
    k = max(1, int(T * _CAPACITY))

    Tt, ns = _TT, _NS
    nsteps = (B * T) // (Tt * ns)
    sub = T // Tt
    xr = x.reshape(B * T, D)
    b2 = b.reshape(1, 1)

    def mk_spec(j):
        return pl.BlockSpec((Tt, D), lambda i: (i * ns + j, 0))

    mask3, w3 = pl.pallas_call(
        functools.partial(_fused_kernel, k=k, nsteps=nsteps, sub=sub, ns=ns),
        grid=(nsteps,),
        in_specs=[mk_spec(j) for j in range(ns)] + [
            pl.BlockSpec((1, D), lambda i: (0, 0)),
            pl.BlockSpec((1, 1), lambda i: (0, 0)),
        ],
        out_specs=(
            pl.BlockSpec((B, sub, Tt), lambda i: (0, 0, 0)),
            pl.BlockSpec((B, sub, Tt), lambda i: (0, 0, 0)),
        ),
        out_shape=(
            jax.ShapeDtypeStruct((B, sub, Tt), jnp.bool_),
            jax.ShapeDtypeStruct((B, sub, Tt), jnp.float32),
        ),
        scratch_shapes=[pltpu.VMEM((B, sub, Tt), jnp.float32)],
    )(*([xr] * ns), W, b2)
    return (mask3.reshape(B, T), w3.reshape(B, T))


# speedup vs baseline: 1.4908x; 1.0008x over previous
import functools

import jax
import jax.numpy as jnp
from jax.experimental import pallas as pl
from jax.experimental.pallas import tpu as pltpu

_CAPACITY = 0.5


def _select(keys, k):
    Bn, S, L = keys.shape

    def count_ge(c):
        return jnp.sum((keys >= c).astype(jnp.int32), axis=(1, 2),
                       keepdims=True)

    thr = jnp.where(count_ge(jnp.int32(0)) >= k,
                    jnp.int32(0), jnp.int32(-(2**31)))
    for hi_bit in range(30, 0, -2):
        q = jnp.int32(1 << (hi_bit - 1))
        d1 = (count_ge(thr + q) >= k).astype(jnp.int32)
        d2 = (count_ge(thr + 2 * q) >= k).astype(jnp.int32)
        d3 = (count_ge(thr + 3 * q) >= k).astype(jnp.int32)
        thr = thr + q * (d1 + d2 + d3)
    thr = jnp.where(count_ge(thr + 1) >= k, thr + 1, thr)

    gt = keys > thr
    eq = keys == thr
    cnt_gt = jnp.sum(gt.astype(jnp.int32), axis=(1, 2), keepdims=True)
    cnt_eq = jnp.sum(eq.astype(jnp.int32), axis=(1, 2), keepdims=True)
    need = k - cnt_gt

    pos = (jax.lax.broadcasted_iota(jnp.int32, (Bn, S, L), 1) * L
           + jax.lax.broadcasted_iota(jnp.int32, (Bn, S, L), 2))
    T = S * L

    def tie_search():
        lo = jnp.zeros((Bn, 1, 1), jnp.int32)
        hi = jnp.full((Bn, 1, 1), T, jnp.int32)
        for _ in range((T.bit_length() + 1) // 2 + 1):
            w = hi - lo
            m1, m2, m3 = lo + w // 4, lo + w // 2, lo + (3 * w) // 4
            c1 = jnp.sum((eq & (pos < m1)).astype(jnp.int32), axis=(1, 2),
                         keepdims=True) >= need
            c2 = jnp.sum((eq & (pos < m2)).astype(jnp.int32), axis=(1, 2),
                         keepdims=True) >= need
            c3 = jnp.sum((eq & (pos < m3)).astype(jnp.int32), axis=(1, 2),
                         keepdims=True) >= need
            hi = jnp.where(c1, m1, jnp.where(c2, m2, jnp.where(c3, m3, hi)))
            lo = jnp.where(~c3, m3, jnp.where(~c2, m2, jnp.where(~c1, m1, lo)))
        return hi

    no_ties = jnp.all(need == cnt_eq)
    hi = jax.lax.cond(no_ties,
                      lambda: jnp.full((Bn, 1, 1), T, jnp.int32),
                      tie_search)
    return gt | (eq & (pos < hi))


_NS = 2
_TT = 512


def _fused_kernel(*refs, k, nsteps, sub, ns):
    x_refs = refs[:ns]
    w_ref, b_ref, mask_ref, wout_ref, key_ref = refs[ns:]
    i = pl.program_id(0)
    for j in range(ns):
        s = jax.lax.dot_general(
            w_ref[...], x_refs[j][...],
            dimension_numbers=(((1,), (1,)), ((), ())),
            preferred_element_type=jnp.float32,
        ) + b_ref[0, 0]
        a = i * ns + j
        wout_ref[a // sub, a % sub, :] = jax.nn.sigmoid(s)[0]
        b32 = jax.lax.bitcast_convert_type(s, jnp.int32)
        mag = b32 & jnp.int32(0x7FFFFFFF)
        key_ref[a // sub, a % sub, :] = jnp.where(
            b32 >= 0, b32, jnp.int32(-1) - mag)[0]

    @pl.when(i == nsteps - 1)
    def _():
        mask_ref[...] = _select(key_ref[...], k)


def kernel(x, W, b):
    B, T, D = x.shape
    k = max(1, int(T * _CAPACITY))

    Tt, ns = _TT, _NS
    nsteps = (B * T) // (Tt * ns)
    sub = T // Tt
    xr = x.reshape(B * T, D)
    b2 = b.reshape(1, 1)

    def mk_spec(j):
        return pl.BlockSpec((Tt, D), lambda i: (i * ns + j, 0))

    mask3, w3 = pl.pallas_call(
        functools.partial(_fused_kernel, k=k, nsteps=nsteps, sub=sub, ns=ns),
        grid=(nsteps,),
        in_specs=[mk_spec(j) for j in range(ns)] + [
            pl.BlockSpec((1, D), lambda i: (0, 0)),
            pl.BlockSpec((1, 1), lambda i: (0, 0)),
        ],
        out_specs=(
            pl.BlockSpec((B, sub, Tt), lambda i: (0, 0, 0)),
            pl.BlockSpec((B, sub, Tt), lambda i: (0, 0, 0)),
        ),
        out_shape=(
            jax.ShapeDtypeStruct((B, sub, Tt), jnp.bool_),
            jax.ShapeDtypeStruct((B, sub, Tt), jnp.float32),
        ),
        scratch_shapes=[pltpu.VMEM((B, sub, Tt), jnp.int32)],
    )(*([xr] * ns), W, b2)
    return (mask3.reshape(B, T), w3.reshape(B, T))
